# baseline (speedup 1.0000x reference)
import jax
import jax.numpy as jnp
from jax import lax
from jax.experimental import pallas as pl
from jax.experimental.pallas import tpu as pltpu
from jax.experimental.pallas import tpu_sc as plsc

MAX_F = 64
_NC, _NS, _L = 2, 16, 16
_NW = _NC * _NS
_SLABS = 256
_SPW = _SLABS // _NW
_CT = 32
_NTC = 512 // _CT
_NB = 3
_NITER = _SPW * _NTC


def _sc_body(x_hbm, f_hbm, t_hbm, o_hbm, freq_r, time_v, in_b, out_b, in_s, out_s):
    c = lax.axis_index("c")
    sid = lax.axis_index("s")
    wid = sid * _NC + c
    slab0 = wid * _SPW
    f_lo = lax.rem(slab0, MAX_F)
    pltpu.sync_copy(f_hbm.at[pl.ds(f_lo, _SPW)], freq_r)

    def in_copy(i, slot):
        slab = slab0 + lax.rem(i, _SPW)
        tc = lax.div(i, _SPW)
        return pltpu.make_async_copy(
            x_hbm.at[slab, pl.ds(tc * _CT, _CT), :], in_b.at[slot], in_s.at[slot]
        )

    def out_copy(i, slot):
        slab = slab0 + lax.rem(i, _SPW)
        tc = lax.div(i, _SPW)
        return pltpu.make_async_copy(
            out_b.at[slot], o_hbm.at[slab, pl.ds(tc * _CT, _CT), :], out_s.at[slot]
        )

    pltpu.sync_copy(t_hbm.at[pl.ds(0, _CT)], time_v)
    for b in range(_NB):
        in_copy(b, b).start()

    def step(i, _):
        slot = lax.rem(i, _NB)
        k = lax.rem(i, _SPW)
        tc = lax.div(i, _SPW)

        @pl.when(jnp.logical_and(k == 0, i > 0))
        def _():
            pltpu.sync_copy(t_hbm.at[pl.ds(tc * _CT, _CT)], time_v)

        in_copy(i, slot).wait()

        @pl.when(i >= _NB)
        def _():
            out_copy(i - _NB, slot).wait()

        fvec = [freq_r[k, pl.ds(j * _L, _L)] for j in range(16)]

        def row(r, _2):
            xa = [in_b[slot, r, pl.ds(j * _L, _L)] for j in range(16)]
            for j in range(16):
                out_b[slot, r, pl.ds(j * _L, _L)] = xa[j] + fvec[j]
            xb = [in_b[slot, r, pl.ds(256 + j * _L, _L)] for j in range(16)]
            tv = [time_v[r, pl.ds(j * _L, _L)] for j in range(16)]
            for j in range(16):
                out_b[slot, r, pl.ds(256 + j * _L, _L)] = xb[j] + tv[j]
            return 0

        lax.fori_loop(0, _CT, row, 0)
        out_copy(i, slot).start()

        @pl.when(i + _NB < _NITER)
        def _():
            in_copy(i + _NB, slot).start()

        return 0

    lax.fori_loop(0, _NITER, step, 0)
    for b in range(_NB):
        i = _NITER - _NB + b
        out_copy(i, lax.rem(i, _NB)).wait()


def kernel(x, freq_embed, time_embed):
    B, F, T, D = x.shape
    xf = x.reshape(B * F, T, D)
    out = pl.kernel(
        _sc_body,
        out_type=jax.ShapeDtypeStruct(xf.shape, x.dtype),
        mesh=plsc.VectorSubcoreMesh(core_axis_name="c", subcore_axis_name="s"),
        scratch_types=[
            pltpu.VMEM((_SPW, 256), jnp.float32),
            pltpu.VMEM((_CT, 256), jnp.float32),
            pltpu.VMEM((_NB, _CT, 512), jnp.float32),
            pltpu.VMEM((_NB, _CT, 512), jnp.float32),
            pltpu.SemaphoreType.DMA((_NB,)),
            pltpu.SemaphoreType.DMA((_NB,)),
        ],
    )(xf, freq_embed, time_embed)
    return out.reshape(B, F, T, D)


# speedup vs baseline: 1.0550x; 1.0550x over previous
import jax
import jax.numpy as jnp
from jax import lax
from jax.experimental import pallas as pl
from jax.experimental.pallas import tpu as pltpu
from jax.experimental.pallas import tpu_sc as plsc

MAX_F = 64
_NC, _NS, _L = 2, 16, 16
_NW = _NC * _NS
_TR = 512 // _NW
_NB = 6
_SLABS = 256


def _sc_body(x_hbm, f_hbm, t_hbm, o_hbm, freq_v, time_v, in_b, out_b, in_s, out_s):
    c = lax.axis_index("c")
    sid = lax.axis_index("s")
    wid = sid * _NC + c
    t_lo = wid * _TR
    pltpu.sync_copy(f_hbm, freq_v)
    pltpu.sync_copy(t_hbm.at[pl.ds(t_lo, _TR)], time_v)

    def in_copy(slab, b):
        return pltpu.make_async_copy(
            x_hbm.at[slab, pl.ds(t_lo, _TR), :], in_b.at[b], in_s.at[b]
        )

    def out_copy(slab, b):
        return pltpu.make_async_copy(
            out_b.at[b], o_hbm.at[slab, pl.ds(t_lo, _TR), :], out_s.at[b]
        )

    for b in range(_NB):
        in_copy(b, b).start()

    def step(slab, _):
        b = lax.rem(slab, _NB)
        in_copy(slab, b).wait()

        @pl.when(slab >= _NB)
        def _():
            out_copy(slab - _NB, b).wait()

        f0 = lax.rem(slab, MAX_F)
        fvec = [freq_v[f0, pl.ds(j * _L, _L)] for j in range(16)]
        for r in range(_TR):
            xa = [in_b[b, r, pl.ds(j * _L, _L)] for j in range(16)]
            for j in range(16):
                out_b[b, r, pl.ds(j * _L, _L)] = xa[j] + fvec[j]
            xb = [in_b[b, r, pl.ds(256 + j * _L, _L)] for j in range(16)]
            tv = [time_v[r, pl.ds(j * _L, _L)] for j in range(16)]
            for j in range(16):
                out_b[b, r, pl.ds(256 + j * _L, _L)] = xb[j] + tv[j]

        out_copy(slab, b).start()

        @pl.when(slab + _NB < _SLABS)
        def _():
            in_copy(slab + _NB, b).start()

        return 0

    lax.fori_loop(0, _SLABS, step, 0)
    for b in range(_NB):
        slab = _SLABS - _NB + b
        out_copy(slab, lax.rem(slab, _NB)).wait()


def kernel(x, freq_embed, time_embed):
    B, F, T, D = x.shape
    xf = x.reshape(B * F, T, D)
    out = pl.kernel(
        _sc_body,
        out_type=jax.ShapeDtypeStruct(xf.shape, x.dtype),
        mesh=plsc.VectorSubcoreMesh(core_axis_name="c", subcore_axis_name="s"),
        scratch_types=[
            pltpu.VMEM((MAX_F, 256), jnp.float32),
            pltpu.VMEM((_TR, 256), jnp.float32),
            pltpu.VMEM((_NB, _TR, 512), jnp.float32),
            pltpu.VMEM((_NB, _TR, 512), jnp.float32),
            pltpu.SemaphoreType.DMA((_NB,)),
            pltpu.SemaphoreType.DMA((_NB,)),
        ],
    )(xf, freq_embed, time_embed)
    return out.reshape(B, F, T, D)
